# baseline (speedup 1.0000x reference)
import functools

import jax
import jax.numpy as jnp
from jax import lax
from jax.experimental import pallas as pl
from jax.experimental.pallas import tpu as pltpu
from jax.experimental.pallas import tpu_sc as plsc

N = 10000
E = 320000
DIN = 128
DH = 64
NC = 2
NS = 16
NW = NC * NS
ROWS128 = 128
K = 80
EP = NW * K * ROWS128
PAD = EP - E
NACC = 10112
ZR = NACC // NS
OUTR = NACC // NS
NOUT = NACC
NBUF = 10
NJUNK = NACC - N
RB = 2000
GRID = N // RB


def _mesh():
    return plsc.VectorSubcoreMesh(
        core_axis_name="c", subcore_axis_name="s", num_cores=NC, num_subcores=NS
    )


def _sc_degree(dst2d, ones16, zeros16):
    @functools.partial(
        pl.kernel,
        out_type=jax.ShapeDtypeStruct((NC, NOUT, 16), jnp.float32),
        mesh=_mesh(),
        scratch_types=[
            pltpu.VMEM((K, ROWS128), jnp.int32),
            pltpu.VMEM((ROWS128, 16), jnp.float32),
            pltpu.VMEM_SHARED((NACC, 16), jnp.float32),
        ],
        compiler_params=pltpu.CompilerParams(use_tc_tiling_on_sc=False),
    )
    def k(dst_hbm, ones_hbm, z_hbm, out_hbm, dst_v, ones_v, acc):
        c = lax.axis_index("c")
        s = lax.axis_index("s")
        wid = c * NS + s
        pltpu.sync_copy(z_hbm, acc.at[pl.ds(s * ZR, ZR)])
        pltpu.sync_copy(dst_hbm.at[pl.ds(wid * K, K)], dst_v)
        pltpu.sync_copy(ones_hbm, ones_v)
        plsc.subcore_barrier()

        def body(j, carry):
            pltpu.sync_copy(ones_v, acc.at[dst_v.at[j]], add=True)
            return carry

        lax.fori_loop(0, K, body, 0)
        plsc.subcore_barrier()
        pltpu.sync_copy(
            acc.at[pl.ds(s * OUTR, OUTR)], out_hbm.at[c, pl.ds(s * OUTR, OUTR), :]
        )

    return k(dst2d, ones16, zeros16)


def _sc_aggregate(g, src2d, dst2d, zeros64):
    @functools.partial(
        pl.kernel,
        out_type=jax.ShapeDtypeStruct((NC, NOUT, DH), jnp.float32),
        mesh=_mesh(),
        scratch_types=[
            pltpu.VMEM((2, NBUF, ROWS128), jnp.int32),
            pltpu.VMEM((2, NBUF, ROWS128), jnp.int32),
            pltpu.VMEM((NBUF, ROWS128, DH), jnp.float32),
            pltpu.VMEM_SHARED((NACC, DH), jnp.float32),
            pltpu.SemaphoreType.DMA((NBUF,)),
            pltpu.SemaphoreType.DMA((NBUF,)),
            pltpu.SemaphoreType.DMA((2,)),
        ],
        compiler_params=pltpu.CompilerParams(use_tc_tiling_on_sc=False),
    )
    def k(g_hbm, src_hbm, dst_hbm, z_hbm, out_hbm, src_v, dst_v,
          rows_v, acc, gsem, ssem, isem):
        c = lax.axis_index("c")
        s = lax.axis_index("s")
        pltpu.sync_copy(z_hbm, acc.at[pl.ds(s * ZR, ZR)])
        plsc.subcore_barrier()

        def gather_start(bank, b):
            pltpu.async_copy(
                g_hbm.at[src_v.at[bank, b]], rows_v.at[b], gsem.at[b]
            )

        def gather_wait(bank, b):
            pltpu.make_async_copy(
                g_hbm.at[src_v.at[bank, b]], rows_v.at[b], gsem.at[b]
            ).wait()

        def scatter_start(bank, b):
            pltpu.async_copy(
                rows_v.at[b], acc.at[dst_v.at[bank, b]], ssem.at[b], add=True
            )

        def scatter_wait(bank, b):
            pltpu.make_async_copy(
                rows_v.at[b], acc.at[dst_v.at[bank, b]], ssem.at[b]
            ).wait()

        def idx_fetch_start(row0, bank):
            pltpu.async_copy(
                src_hbm.at[pl.ds(row0, NBUF)], src_v.at[bank], isem.at[0]
            )
            pltpu.async_copy(
                dst_hbm.at[pl.ds(row0, NBUF)], dst_v.at[bank], isem.at[1]
            )

        def idx_fetch_wait(row0, bank):
            pltpu.make_async_copy(
                src_hbm.at[pl.ds(row0, NBUF)], src_v.at[bank], isem.at[0]
            ).wait()
            pltpu.make_async_copy(
                dst_hbm.at[pl.ds(row0, NBUF)], dst_v.at[bank], isem.at[1]
            ).wait()

        def run(kc, base):
            nsc = kc // NBUF
            idx_fetch_start(base, 0)
            idx_fetch_wait(base, 0)
            for b in range(NBUF):
                gather_start(0, b)

            def body(sc_i, carry):
                row_next = base + (sc_i + 1) * NBUF
                bank = lax.rem(sc_i, 2)
                nbank = lax.rem(sc_i + 1, 2)
                more = sc_i + 1 < nsc

                @pl.when(more)
                def _():
                    idx_fetch_start(row_next, nbank)

                for b in range(NBUF):
                    gather_wait(bank, b)
                    scatter_start(bank, b)

                @pl.when(more)
                def _():
                    idx_fetch_wait(row_next, nbank)

                for b in range(NBUF):
                    scatter_wait(bank, b)

                    @pl.when(more)
                    def _():
                        gather_start(nbank, b)

                return carry

            lax.fori_loop(0, nsc, body, 0)

        run(K, (c * NS + s) * K)

        plsc.subcore_barrier()
        pltpu.sync_copy(
            acc.at[pl.ds(s * OUTR, OUTR)], out_hbm.at[c, pl.ds(s * OUTR, OUTR), :]
        )

    return k(g, src2d, dst2d, zeros64)


def _dinv_block(degp_ref):
    deg = 1.0 + degp_ref[0, :, 0:1] + degp_ref[1, :, 0:1]
    return lax.rsqrt(deg)


def _tc_stage1(x, W1, degp):
    def body(x_ref, w_ref, degp_ref, h_ref, g_ref):
        h = jnp.dot(x_ref[...], w_ref[...], preferred_element_type=jnp.float32)
        h_ref[...] = h
        g_ref[...] = _dinv_block(degp_ref) * h

    return pl.pallas_call(
        body,
        grid=(GRID,),
        in_specs=[
            pl.BlockSpec((RB, DIN), lambda i: (i, 0)),
            pl.BlockSpec((DIN, DH), lambda i: (0, 0)),
            pl.BlockSpec((NC, RB, 16), lambda i: (0, i, 0)),
        ],
        out_specs=[
            pl.BlockSpec((RB, DH), lambda i: (i, 0)),
            pl.BlockSpec((RB, DH), lambda i: (i, 0)),
        ],
        out_shape=[
            jax.ShapeDtypeStruct((N, DH), jnp.float32),
            jax.ShapeDtypeStruct((N, DH), jnp.float32),
        ],
    )(x, W1, degp)


def _tc_stage2(aggp, h1pre, degp, b1, W2):
    def body(aggp_ref, hpre_ref, degp_ref, b_ref, w_ref, h_ref, g_ref):
        dinv = _dinv_block(degp_ref)
        agg = aggp_ref[0] + aggp_ref[1]
        h1 = jnp.maximum(
            dinv * agg + (dinv * dinv) * hpre_ref[...] + b_ref[...], 0.0
        )
        h2 = jnp.dot(h1, w_ref[...], preferred_element_type=jnp.float32)
        h_ref[...] = h2
        g_ref[...] = dinv * h2

    return pl.pallas_call(
        body,
        grid=(GRID,),
        in_specs=[
            pl.BlockSpec((NC, RB, DH), lambda i: (0, i, 0)),
            pl.BlockSpec((RB, DH), lambda i: (i, 0)),
            pl.BlockSpec((NC, RB, 16), lambda i: (0, i, 0)),
            pl.BlockSpec((1, DH), lambda i: (0, 0)),
            pl.BlockSpec((DH, DH), lambda i: (0, 0)),
        ],
        out_specs=[
            pl.BlockSpec((RB, DH), lambda i: (i, 0)),
            pl.BlockSpec((RB, DH), lambda i: (i, 0)),
        ],
        out_shape=[
            jax.ShapeDtypeStruct((N, DH), jnp.float32),
            jax.ShapeDtypeStruct((N, DH), jnp.float32),
        ],
    )(aggp, h1pre, degp, b1, W2)


def _tc_stage3(aggp, h2pre, degp, b2, Wout, bout):
    def body(aggp_ref, hpre_ref, degp_ref, b_ref, w_ref, bo_ref, o_ref):
        dinv = _dinv_block(degp_ref)
        agg = aggp_ref[0] + aggp_ref[1]
        h2 = jnp.maximum(
            dinv * agg + (dinv * dinv) * hpre_ref[...] + b_ref[...], 0.0
        )
        o_ref[...] = (
            jnp.dot(h2, w_ref[...], preferred_element_type=jnp.float32)
            + bo_ref[...]
        )

    return pl.pallas_call(
        body,
        grid=(GRID,),
        in_specs=[
            pl.BlockSpec((NC, RB, DH), lambda i: (0, i, 0)),
            pl.BlockSpec((RB, DH), lambda i: (i, 0)),
            pl.BlockSpec((NC, RB, 16), lambda i: (0, i, 0)),
            pl.BlockSpec((1, DH), lambda i: (0, 0)),
            pl.BlockSpec((DH, 1), lambda i: (0, 0)),
            pl.BlockSpec((1, 1), lambda i: (0, 0)),
        ],
        out_specs=pl.BlockSpec((RB, 1), lambda i: (i, 0)),
        out_shape=jax.ShapeDtypeStruct((N, 1), jnp.float32),
    )(aggp, h2pre, degp, b2, Wout, bout)


def kernel(x, edge_index, edge_attr, W1, b1, W2, b2, Wout, bout):
    del edge_attr
    src = edge_index[0]
    dst = edge_index[1]
    pad_iota = jnp.arange(PAD, dtype=jnp.int32)
    src_p = jnp.concatenate([src, pad_iota % N])
    dst_p = jnp.concatenate([dst, N + pad_iota % NJUNK])
    src2d = src_p.reshape(NW * K, ROWS128)
    dst2d = dst_p.reshape(NW * K, ROWS128)
    ones16 = jnp.ones((ROWS128, 16), jnp.float32)
    zeros16 = jnp.zeros((ZR, 16), jnp.float32)
    zeros64 = jnp.zeros((ZR, DH), jnp.float32)

    degp = _sc_degree(dst2d, ones16, zeros16)
    h1pre, g1 = _tc_stage1(x, W1, degp)
    agg1 = _sc_aggregate(g1, src2d, dst2d, zeros64)
    h2pre, g2 = _tc_stage2(agg1, h1pre, degp, b1.reshape(1, DH), W2)
    agg2 = _sc_aggregate(g2, src2d, dst2d, zeros64)
    out = _tc_stage3(
        agg2, h2pre, degp, b2.reshape(1, DH), Wout, bout.reshape(1, 1)
    )
    return out


# speedup vs baseline: 1.0065x; 1.0065x over previous
import functools

import jax
import jax.numpy as jnp
from jax import lax
from jax.experimental import pallas as pl
from jax.experimental.pallas import tpu as pltpu
from jax.experimental.pallas import tpu_sc as plsc

N = 10000
E = 320000
DIN = 128
DH = 64
NC = 2
NS = 16
NW = NC * NS
ROWS128 = 128
K = 80
EP = NW * K * ROWS128
PAD = EP - E
NACC = 10112
ZR = NACC // NS
OUTR = NACC // NS
NOUT = NACC
NBUF = 10
NJUNK = NACC - N
RB = 2000
GRID = N // RB


def _mesh():
    return plsc.VectorSubcoreMesh(
        core_axis_name="c", subcore_axis_name="s", num_cores=NC, num_subcores=NS
    )


def _sc_degree(dst2d, ones16, zeros16):
    @functools.partial(
        pl.kernel,
        out_type=jax.ShapeDtypeStruct((NC, NOUT, 16), jnp.float32),
        mesh=_mesh(),
        scratch_types=[
            pltpu.VMEM((K, ROWS128), jnp.int32),
            pltpu.VMEM((ROWS128, 16), jnp.float32),
            pltpu.VMEM_SHARED((NACC, 16), jnp.float32),
        ],
        compiler_params=pltpu.CompilerParams(use_tc_tiling_on_sc=False),
    )
    def k(dst_hbm, ones_hbm, z_hbm, out_hbm, dst_v, ones_v, acc):
        c = lax.axis_index("c")
        s = lax.axis_index("s")
        wid = c * NS + s
        pltpu.sync_copy(z_hbm, acc.at[pl.ds(s * ZR, ZR)])
        pltpu.sync_copy(dst_hbm.at[pl.ds(wid * K, K)], dst_v)
        pltpu.sync_copy(ones_hbm, ones_v)
        plsc.subcore_barrier()

        def body(j, carry):
            pltpu.sync_copy(ones_v, acc.at[dst_v.at[j]], add=True)
            return carry

        lax.fori_loop(0, K, body, 0)
        plsc.subcore_barrier()
        pltpu.sync_copy(
            acc.at[pl.ds(s * OUTR, OUTR)], out_hbm.at[c, pl.ds(s * OUTR, OUTR), :]
        )

    return k(dst2d, ones16, zeros16)


def _sc_aggregate(g, src2d, dst2d, zeros64):
    @functools.partial(
        pl.kernel,
        out_type=jax.ShapeDtypeStruct((NC, NOUT, DH), jnp.float32),
        mesh=_mesh(),
        scratch_types=[
            pltpu.VMEM((2, NBUF, ROWS128), jnp.int32),
            pltpu.VMEM((2, NBUF, ROWS128), jnp.int32),
            pltpu.VMEM((NBUF, ROWS128, DH), jnp.float32),
            pltpu.VMEM_SHARED((NACC, DH), jnp.float32),
            pltpu.SemaphoreType.DMA((NBUF,)),
            pltpu.SemaphoreType.DMA((NBUF,)),
            pltpu.SemaphoreType.DMA((2,)),
        ],
        compiler_params=pltpu.CompilerParams(use_tc_tiling_on_sc=False),
    )
    def k(g_hbm, src_hbm, dst_hbm, z_hbm, out_hbm, src_v, dst_v,
          rows_v, acc, gsem, ssem, isem):
        c = lax.axis_index("c")
        s = lax.axis_index("s")
        pltpu.sync_copy(z_hbm, acc.at[pl.ds(s * ZR, ZR)])
        plsc.subcore_barrier()

        def gather_start(bank, b):
            pltpu.async_copy(
                g_hbm.at[src_v.at[bank, b]], rows_v.at[b], gsem.at[b]
            )

        def gather_wait(bank, b):
            pltpu.make_async_copy(
                g_hbm.at[src_v.at[bank, b]], rows_v.at[b], gsem.at[b]
            ).wait()

        def scatter_start(bank, b):
            pltpu.async_copy(
                rows_v.at[b], acc.at[dst_v.at[bank, b]], ssem.at[b], add=True
            )

        def scatter_wait(bank, b):
            pltpu.make_async_copy(
                rows_v.at[b], acc.at[dst_v.at[bank, b]], ssem.at[b]
            ).wait()

        def idx_fetch_start(row0, bank):
            pltpu.async_copy(
                src_hbm.at[pl.ds(row0, NBUF)], src_v.at[bank], isem.at[0]
            )
            pltpu.async_copy(
                dst_hbm.at[pl.ds(row0, NBUF)], dst_v.at[bank], isem.at[1]
            )

        def idx_fetch_wait(row0, bank):
            pltpu.make_async_copy(
                src_hbm.at[pl.ds(row0, NBUF)], src_v.at[bank], isem.at[0]
            ).wait()
            pltpu.make_async_copy(
                dst_hbm.at[pl.ds(row0, NBUF)], dst_v.at[bank], isem.at[1]
            ).wait()

        def run(kc, base):
            nsc = kc // NBUF
            idx_fetch_start(base, 0)
            idx_fetch_wait(base, 0)
            for b in range(NBUF):
                gather_start(0, b)

            def body(sc_i, carry):
                row_next = base + (sc_i + 1) * NBUF
                bank = lax.rem(sc_i, 2)
                nbank = lax.rem(sc_i + 1, 2)
                more = sc_i + 1 < nsc

                @pl.when(more)
                def _():
                    idx_fetch_start(row_next, nbank)

                for b in range(NBUF):
                    gather_wait(bank, b)
                    scatter_start(bank, b)

                @pl.when(more)
                def _():
                    idx_fetch_wait(row_next, nbank)

                for b in range(NBUF):
                    scatter_wait(bank, b)

                    @pl.when(more)
                    def _():
                        gather_start(nbank, b)

                return carry

            lax.fori_loop(0, nsc, body, 0)

        run(K, (c * NS + s) * K)

        plsc.subcore_barrier()
        pltpu.sync_copy(
            acc.at[pl.ds(s * OUTR, OUTR)], out_hbm.at[c, pl.ds(s * OUTR, OUTR), :]
        )

    return k(g, src2d, dst2d, zeros64)


def _dinv_block(degp_ref):
    deg = 1.0 + degp_ref[0, :, 0:1] + degp_ref[1, :, 0:1]
    return lax.rsqrt(deg)


def _tc_matmul1(x, W1):
    def body(x_ref, w_ref, h_ref):
        h_ref[...] = jnp.dot(
            x_ref[...], w_ref[...], preferred_element_type=jnp.float32
        )

    return pl.pallas_call(
        body,
        grid=(GRID,),
        in_specs=[
            pl.BlockSpec((RB, DIN), lambda i: (i, 0)),
            pl.BlockSpec((DIN, DH), lambda i: (0, 0)),
        ],
        out_specs=pl.BlockSpec((RB, DH), lambda i: (i, 0)),
        out_shape=jax.ShapeDtypeStruct((N, DH), jnp.float32),
    )(x, W1)


def _tc_stage1(h1pre, degp):
    def body(h_ref, degp_ref, g_ref):
        g_ref[...] = _dinv_block(degp_ref) * h_ref[...]

    return pl.pallas_call(
        body,
        grid=(GRID,),
        in_specs=[
            pl.BlockSpec((RB, DH), lambda i: (i, 0)),
            pl.BlockSpec((NC, RB, 16), lambda i: (0, i, 0)),
        ],
        out_specs=pl.BlockSpec((RB, DH), lambda i: (i, 0)),
        out_shape=jax.ShapeDtypeStruct((N, DH), jnp.float32),
    )(h1pre, degp)


def _tc_stage2(aggp, h1pre, degp, b1, W2):
    def body(aggp_ref, hpre_ref, degp_ref, b_ref, w_ref, h_ref, g_ref):
        dinv = _dinv_block(degp_ref)
        agg = aggp_ref[0] + aggp_ref[1]
        h1 = jnp.maximum(
            dinv * agg + (dinv * dinv) * hpre_ref[...] + b_ref[...], 0.0
        )
        h2 = jnp.dot(h1, w_ref[...], preferred_element_type=jnp.float32)
        h_ref[...] = h2
        g_ref[...] = dinv * h2

    return pl.pallas_call(
        body,
        grid=(GRID,),
        in_specs=[
            pl.BlockSpec((NC, RB, DH), lambda i: (0, i, 0)),
            pl.BlockSpec((RB, DH), lambda i: (i, 0)),
            pl.BlockSpec((NC, RB, 16), lambda i: (0, i, 0)),
            pl.BlockSpec((1, DH), lambda i: (0, 0)),
            pl.BlockSpec((DH, DH), lambda i: (0, 0)),
        ],
        out_specs=[
            pl.BlockSpec((RB, DH), lambda i: (i, 0)),
            pl.BlockSpec((RB, DH), lambda i: (i, 0)),
        ],
        out_shape=[
            jax.ShapeDtypeStruct((N, DH), jnp.float32),
            jax.ShapeDtypeStruct((N, DH), jnp.float32),
        ],
    )(aggp, h1pre, degp, b1, W2)


def _tc_stage3(aggp, h2pre, degp, b2, Wout, bout):
    def body(aggp_ref, hpre_ref, degp_ref, b_ref, w_ref, bo_ref, o_ref):
        dinv = _dinv_block(degp_ref)
        agg = aggp_ref[0] + aggp_ref[1]
        h2 = jnp.maximum(
            dinv * agg + (dinv * dinv) * hpre_ref[...] + b_ref[...], 0.0
        )
        o_ref[...] = (
            jnp.dot(h2, w_ref[...], preferred_element_type=jnp.float32)
            + bo_ref[...]
        )

    return pl.pallas_call(
        body,
        grid=(GRID,),
        in_specs=[
            pl.BlockSpec((NC, RB, DH), lambda i: (0, i, 0)),
            pl.BlockSpec((RB, DH), lambda i: (i, 0)),
            pl.BlockSpec((NC, RB, 16), lambda i: (0, i, 0)),
            pl.BlockSpec((1, DH), lambda i: (0, 0)),
            pl.BlockSpec((DH, 1), lambda i: (0, 0)),
            pl.BlockSpec((1, 1), lambda i: (0, 0)),
        ],
        out_specs=pl.BlockSpec((RB, 1), lambda i: (i, 0)),
        out_shape=jax.ShapeDtypeStruct((N, 1), jnp.float32),
    )(aggp, h2pre, degp, b2, Wout, bout)


def kernel(x, edge_index, edge_attr, W1, b1, W2, b2, Wout, bout):
    del edge_attr
    src = edge_index[0]
    dst = edge_index[1]
    pad_iota = jnp.arange(PAD, dtype=jnp.int32)
    src_p = jnp.concatenate([src, pad_iota % N])
    dst_p = jnp.concatenate([dst, N + pad_iota % NJUNK])
    src2d = src_p.reshape(NW * K, ROWS128)
    dst2d = dst_p.reshape(NW * K, ROWS128)
    ones16 = jnp.ones((ROWS128, 16), jnp.float32)
    zeros16 = jnp.zeros((ZR, 16), jnp.float32)
    zeros64 = jnp.zeros((ZR, DH), jnp.float32)

    h1pre = _tc_matmul1(x, W1)
    degp = _sc_degree(dst2d, ones16, zeros16)
    g1 = _tc_stage1(h1pre, degp)
    agg1 = _sc_aggregate(g1, src2d, dst2d, zeros64)
    h2pre, g2 = _tc_stage2(agg1, h1pre, degp, b1.reshape(1, DH), W2)
    agg2 = _sc_aggregate(g2, src2d, dst2d, zeros64)
    out = _tc_stage3(
        agg2, h2pre, degp, b2.reshape(1, DH), Wout, bout.reshape(1, 1)
    )
    return out


# speedup vs baseline: 1.0090x; 1.0025x over previous
import functools

import jax
import jax.numpy as jnp
from jax import lax
from jax.experimental import pallas as pl
from jax.experimental.pallas import tpu as pltpu
from jax.experimental.pallas import tpu_sc as plsc

N = 10000
E = 320000
DIN = 128
DH = 64
NC = 2
NS = 16
NW = NC * NS
ROWS128 = 128
K = 80
EP = NW * K * ROWS128
PAD = EP - E
NACC = 10112
ZR = NACC // NS
OUTR = NACC // NS
NOUT = NACC
NBUF = 10
NJUNK = NACC - N
RB = 2000
GRID = N // RB
DEGW = 8


def _mesh():
    return plsc.VectorSubcoreMesh(
        core_axis_name="c", subcore_axis_name="s", num_cores=NC, num_subcores=NS
    )


def _sc_degree(dst2d, ones16, zeros16):
    @functools.partial(
        pl.kernel,
        out_type=jax.ShapeDtypeStruct((NC, NOUT, DEGW), jnp.float32),
        mesh=_mesh(),
        scratch_types=[
            pltpu.VMEM((K, ROWS128), jnp.int32),
            pltpu.VMEM((ROWS128, DEGW), jnp.float32),
            pltpu.VMEM_SHARED((NACC, DEGW), jnp.float32),
        ],
        compiler_params=pltpu.CompilerParams(use_tc_tiling_on_sc=False),
    )
    def k(dst_hbm, ones_hbm, z_hbm, out_hbm, dst_v, ones_v, acc):
        c = lax.axis_index("c")
        s = lax.axis_index("s")
        wid = c * NS + s
        pltpu.sync_copy(z_hbm, acc.at[pl.ds(s * ZR, ZR)])
        pltpu.sync_copy(dst_hbm.at[pl.ds(wid * K, K)], dst_v)
        pltpu.sync_copy(ones_hbm, ones_v)
        plsc.subcore_barrier()

        def body(j, carry):
            pltpu.sync_copy(ones_v, acc.at[dst_v.at[j]], add=True)
            return carry

        lax.fori_loop(0, K, body, 0)
        plsc.subcore_barrier()
        pltpu.sync_copy(
            acc.at[pl.ds(s * OUTR, OUTR)], out_hbm.at[c, pl.ds(s * OUTR, OUTR), :]
        )

    return k(dst2d, ones16, zeros16)


def _sc_aggregate(g, src2d, dst2d, zeros64):
    @functools.partial(
        pl.kernel,
        out_type=jax.ShapeDtypeStruct((NC, NOUT, DH), jnp.float32),
        mesh=_mesh(),
        scratch_types=[
            pltpu.VMEM((2, NBUF, ROWS128), jnp.int32),
            pltpu.VMEM((2, NBUF, ROWS128), jnp.int32),
            pltpu.VMEM((NBUF, ROWS128, DH), jnp.float32),
            pltpu.VMEM_SHARED((NACC, DH), jnp.float32),
            pltpu.SemaphoreType.DMA((NBUF,)),
            pltpu.SemaphoreType.DMA((NBUF,)),
            pltpu.SemaphoreType.DMA((2,)),
        ],
        compiler_params=pltpu.CompilerParams(use_tc_tiling_on_sc=False),
    )
    def k(g_hbm, src_hbm, dst_hbm, z_hbm, out_hbm, src_v, dst_v,
          rows_v, acc, gsem, ssem, isem):
        c = lax.axis_index("c")
        s = lax.axis_index("s")
        pltpu.sync_copy(z_hbm, acc.at[pl.ds(s * ZR, ZR)])
        plsc.subcore_barrier()

        def gather_start(bank, b):
            pltpu.async_copy(
                g_hbm.at[src_v.at[bank, b]], rows_v.at[b], gsem.at[b]
            )

        def gather_wait(bank, b):
            pltpu.make_async_copy(
                g_hbm.at[src_v.at[bank, b]], rows_v.at[b], gsem.at[b]
            ).wait()

        def scatter_start(bank, b):
            pltpu.async_copy(
                rows_v.at[b], acc.at[dst_v.at[bank, b]], ssem.at[b], add=True
            )

        def scatter_wait(bank, b):
            pltpu.make_async_copy(
                rows_v.at[b], acc.at[dst_v.at[bank, b]], ssem.at[b]
            ).wait()

        def idx_fetch_start(row0, bank):
            pltpu.async_copy(
                src_hbm.at[pl.ds(row0, NBUF)], src_v.at[bank], isem.at[0]
            )
            pltpu.async_copy(
                dst_hbm.at[pl.ds(row0, NBUF)], dst_v.at[bank], isem.at[1]
            )

        def idx_fetch_wait(row0, bank):
            pltpu.make_async_copy(
                src_hbm.at[pl.ds(row0, NBUF)], src_v.at[bank], isem.at[0]
            ).wait()
            pltpu.make_async_copy(
                dst_hbm.at[pl.ds(row0, NBUF)], dst_v.at[bank], isem.at[1]
            ).wait()

        def run(kc, base):
            nsc = kc // NBUF
            idx_fetch_start(base, 0)
            idx_fetch_wait(base, 0)
            for b in range(NBUF):
                gather_start(0, b)

            def body(sc_i, carry):
                row_next = base + (sc_i + 1) * NBUF
                bank = lax.rem(sc_i, 2)
                nbank = lax.rem(sc_i + 1, 2)
                more = sc_i + 1 < nsc

                @pl.when(more)
                def _():
                    idx_fetch_start(row_next, nbank)

                for b in range(NBUF):
                    gather_wait(bank, b)
                    scatter_start(bank, b)

                @pl.when(more)
                def _():
                    idx_fetch_wait(row_next, nbank)

                for b in range(NBUF):
                    scatter_wait(bank, b)

                    @pl.when(more)
                    def _():
                        gather_start(nbank, b)

                return carry

            lax.fori_loop(0, nsc, body, 0)

        run(K, (c * NS + s) * K)

        plsc.subcore_barrier()
        pltpu.sync_copy(
            acc.at[pl.ds(s * OUTR, OUTR)], out_hbm.at[c, pl.ds(s * OUTR, OUTR), :]
        )

    return k(g, src2d, dst2d, zeros64)


def _dinv_block(degp_ref):
    deg = 1.0 + degp_ref[0, :, 0:1] + degp_ref[1, :, 0:1]
    return lax.rsqrt(deg)


def _tc_matmul1(x, W1):
    def body(x_ref, w_ref, h_ref):
        h_ref[...] = jnp.dot(
            x_ref[...], w_ref[...], preferred_element_type=jnp.float32
        )

    return pl.pallas_call(
        body,
        grid=(GRID,),
        in_specs=[
            pl.BlockSpec((RB, DIN), lambda i: (i, 0)),
            pl.BlockSpec((DIN, DH), lambda i: (0, 0)),
        ],
        out_specs=pl.BlockSpec((RB, DH), lambda i: (i, 0)),
        out_shape=jax.ShapeDtypeStruct((N, DH), jnp.float32),
    )(x, W1)


def _tc_stage1(h1pre, degp):
    def body(h_ref, degp_ref, g_ref):
        g_ref[...] = _dinv_block(degp_ref) * h_ref[...]

    return pl.pallas_call(
        body,
        grid=(GRID,),
        in_specs=[
            pl.BlockSpec((RB, DH), lambda i: (i, 0)),
            pl.BlockSpec((NC, RB, DEGW), lambda i: (0, i, 0)),
        ],
        out_specs=pl.BlockSpec((RB, DH), lambda i: (i, 0)),
        out_shape=jax.ShapeDtypeStruct((N, DH), jnp.float32),
    )(h1pre, degp)


def _tc_stage2(aggp, h1pre, degp, b1, W2):
    def body(aggp_ref, hpre_ref, degp_ref, b_ref, w_ref, h_ref, g_ref):
        dinv = _dinv_block(degp_ref)
        agg = aggp_ref[0] + aggp_ref[1]
        h1 = jnp.maximum(
            dinv * agg + (dinv * dinv) * hpre_ref[...] + b_ref[...], 0.0
        )
        h2 = jnp.dot(h1, w_ref[...], preferred_element_type=jnp.float32)
        h_ref[...] = h2
        g_ref[...] = dinv * h2

    return pl.pallas_call(
        body,
        grid=(GRID,),
        in_specs=[
            pl.BlockSpec((NC, RB, DH), lambda i: (0, i, 0)),
            pl.BlockSpec((RB, DH), lambda i: (i, 0)),
            pl.BlockSpec((NC, RB, DEGW), lambda i: (0, i, 0)),
            pl.BlockSpec((1, DH), lambda i: (0, 0)),
            pl.BlockSpec((DH, DH), lambda i: (0, 0)),
        ],
        out_specs=[
            pl.BlockSpec((RB, DH), lambda i: (i, 0)),
            pl.BlockSpec((RB, DH), lambda i: (i, 0)),
        ],
        out_shape=[
            jax.ShapeDtypeStruct((N, DH), jnp.float32),
            jax.ShapeDtypeStruct((N, DH), jnp.float32),
        ],
    )(aggp, h1pre, degp, b1, W2)


def _tc_stage3(aggp, h2pre, degp, b2, Wout, bout):
    def body(aggp_ref, hpre_ref, degp_ref, b_ref, w_ref, bo_ref, o_ref):
        dinv = _dinv_block(degp_ref)
        agg = aggp_ref[0] + aggp_ref[1]
        h2 = jnp.maximum(
            dinv * agg + (dinv * dinv) * hpre_ref[...] + b_ref[...], 0.0
        )
        o_ref[...] = (
            jnp.dot(h2, w_ref[...], preferred_element_type=jnp.float32)
            + bo_ref[...]
        )

    return pl.pallas_call(
        body,
        grid=(GRID,),
        in_specs=[
            pl.BlockSpec((NC, RB, DH), lambda i: (0, i, 0)),
            pl.BlockSpec((RB, DH), lambda i: (i, 0)),
            pl.BlockSpec((NC, RB, DEGW), lambda i: (0, i, 0)),
            pl.BlockSpec((1, DH), lambda i: (0, 0)),
            pl.BlockSpec((DH, 1), lambda i: (0, 0)),
            pl.BlockSpec((1, 1), lambda i: (0, 0)),
        ],
        out_specs=pl.BlockSpec((RB, 1), lambda i: (i, 0)),
        out_shape=jax.ShapeDtypeStruct((N, 1), jnp.float32),
    )(aggp, h2pre, degp, b2, Wout, bout)


def kernel(x, edge_index, edge_attr, W1, b1, W2, b2, Wout, bout):
    del edge_attr
    src = edge_index[0]
    dst = edge_index[1]
    pad_iota = jnp.arange(PAD, dtype=jnp.int32)
    src_p = jnp.concatenate([src, pad_iota % N])
    dst_p = jnp.concatenate([dst, N + pad_iota % NJUNK])
    src2d = src_p.reshape(NW * K, ROWS128)
    dst2d = dst_p.reshape(NW * K, ROWS128)
    ones16 = jnp.ones((ROWS128, DEGW), jnp.float32)
    zeros16 = jnp.zeros((ZR, DEGW), jnp.float32)
    zeros64 = jnp.zeros((ZR, DH), jnp.float32)

    h1pre = _tc_matmul1(x, W1)
    degp = _sc_degree(dst2d, ones16, zeros16)
    g1 = _tc_stage1(h1pre, degp)
    agg1 = _sc_aggregate(g1, src2d, dst2d, zeros64)
    h2pre, g2 = _tc_stage2(agg1, h1pre, degp, b1.reshape(1, DH), W2)
    agg2 = _sc_aggregate(g2, src2d, dst2d, zeros64)
    out = _tc_stage3(
        agg2, h2pre, degp, b2.reshape(1, DH), Wout, bout.reshape(1, 1)
    )
    return out
